# baseline (speedup 1.0000x reference)
import functools

import jax
import jax.numpy as jnp
from jax import lax
from jax.experimental import pallas as pl
from jax.experimental.pallas import tpu as pltpu
from jax.experimental.pallas import tpu_sc as plsc

NC = 2
NS = 16
NW = NC * NS
L = 16
CH = 128


def _make_kernel(B, D):
    b_per_w = B // NW
    n_ch = b_per_w // CH
    n_grp = CH // L
    assert n_ch % 2 == 0
    mesh = plsc.VectorSubcoreMesh(core_axis_name="c", subcore_axis_name="s")
    idx_t = pltpu.VMEM((n_ch, CH), jnp.int32)
    buf_t = pltpu.VMEM((CH, 2 * D), jnp.float32)

    @functools.partial(
        pl.kernel,
        mesh=mesh,
        compiler_params=pltpu.CompilerParams(needs_layout_passes=False),
        out_type=jax.ShapeDtypeStruct((B,), jnp.float32),
        scratch_types=[
            idx_t, idx_t, idx_t,
            idx_t, idx_t, idx_t,
            buf_t, buf_t, buf_t,
            buf_t, buf_t, buf_t,
            pltpu.VMEM((b_per_w,), jnp.float32),
            pltpu.SemaphoreType.DMA,
            pltpu.SemaphoreType.DMA,
        ],
    )
    def k(node_hbm, rel_hbm, srow_hbm, rrow_hbm, orow_hbm,
          shalf_hbm, rhalf_hbm, ohalf_hbm, out_hbm,
          srow_v, rrow_v, orow_v, shalf_v, rhalf_v, ohalf_v,
          sA, rA, oA, sB, rB, oB, out_v, semA, semB):
        wid = lax.axis_index("s") * NC + lax.axis_index("c")
        crow = wid * n_ch
        for hbm, vm in ((srow_hbm, srow_v), (rrow_hbm, rrow_v),
                        (orow_hbm, orow_v), (shalf_hbm, shalf_v),
                        (rhalf_hbm, rhalf_v), (ohalf_hbm, ohalf_v)):
            pltpu.sync_copy(hbm.at[pl.ds(crow, n_ch)], vm)

        lanes = lax.iota(jnp.int32, L)

        def fire(c, sbuf, rbuf, obuf, sem):
            cps = [
                pltpu.async_copy(node_hbm.at[srow_v.at[c]], sbuf, sem),
                pltpu.async_copy(rel_hbm.at[rrow_v.at[c]], rbuf, sem),
                pltpu.async_copy(node_hbm.at[orow_v.at[c]], obuf, sem),
            ]
            return cps

        def drain(sbuf, rbuf, obuf, sem):
            pltpu.make_async_copy(node_hbm.at[pl.ds(0, CH)], sbuf, sem).wait()
            pltpu.make_async_copy(rel_hbm.at[pl.ds(0, CH)], rbuf, sem).wait()
            pltpu.make_async_copy(node_hbm.at[pl.ds(0, CH)], obuf, sem).wait()

        def compute(c, sbuf, rbuf, obuf):
            def grp(g, carry):
                rows = g * L + lanes
                cs = shalf_v[c, pl.ds(g * L, L)] * D
                cr = rhalf_v[c, pl.ds(g * L, L)] * D
                co = ohalf_v[c, pl.ds(g * L, L)] * D
                accs = [jnp.zeros((L,), jnp.float32) for _ in range(4)]
                for d in range(D):
                    sv = plsc.load_gather(sbuf, [rows, cs + d])
                    rv = plsc.load_gather(rbuf, [rows, cr + d])
                    ov = plsc.load_gather(obuf, [rows, co + d])
                    accs[d % 4] = accs[d % 4] + sv * rv * ov
                out_v[pl.ds(c * CH + g * L, L)] = (
                    (accs[0] + accs[1]) + (accs[2] + accs[3]))
                return carry

            lax.fori_loop(0, n_grp, grp, 0)

        fire(0, sA, rA, oA, semA)

        def outer(h, carry):
            g = h * 2
            fire(g + 1, sB, rB, oB, semB)
            drain(sA, rA, oA, semA)
            compute(g, sA, rA, oA)

            @pl.when(g + 2 < n_ch)
            def _():
                fire(g + 2, sA, rA, oA, semA)

            drain(sB, rB, oB, semB)
            compute(g + 1, sB, rB, oB)
            return carry

        lax.fori_loop(0, n_ch // 2, outer, 0)
        pltpu.sync_copy(out_v, out_hbm.at[pl.ds(wid * b_per_w, b_per_w)])

    return k


def kernel(node_embeddings, rel_embeddings, triplets):
    B = triplets.shape[0]
    V, D = node_embeddings.shape
    R = rel_embeddings.shape[0]
    idx = triplets.astype(jnp.int32)
    node2 = node_embeddings.reshape(V // 2, 2 * D)
    rel2 = rel_embeddings.reshape(R // 2, 2 * D)
    rows = lax.shift_right_logical(idx, 1).reshape(B // CH, CH, 3)
    halfs = lax.bitwise_and(idx, 1).reshape(B // CH, CH, 3)
    return _make_kernel(B, D)(
        node2, rel2,
        rows[:, :, 0], rows[:, :, 1], rows[:, :, 2],
        halfs[:, :, 0], halfs[:, :, 1], halfs[:, :, 2],
    )
